# baseline (speedup 1.0000x reference)
import functools

import jax
import jax.numpy as jnp
from jax import lax
from jax.experimental import pallas as pl
from jax.experimental.pallas import tpu as pltpu
from jax.experimental.pallas import tpu_sc as plsc

VOCAB = 1000000
DIM = 64
BATCH = 16384
HIST = 50

NC = 2
NS = 16
NW = NC * NS
N = BATCH * HIST
PER_W = N // NW
CH = 128
UNITS = PER_W // CH
BH = BATCH // CH
DH = DIM // 8
NBUF = 2


@functools.partial(
    pl.kernel,
    out_type=jax.ShapeDtypeStruct((HIST, DH, BH, 8, CH), jnp.float32),
    mesh=plsc.VectorSubcoreMesh(
        core_axis_name="c", subcore_axis_name="s",
        num_cores=NC, num_subcores=NS),
    scratch_types=[
        pltpu.VMEM((PER_W,), jnp.int32),
        pltpu.VMEM((CH, DIM), jnp.int32),
        pltpu.VMEM((CH, DIM), jnp.int32),
        pltpu.VMEM((CH,), jnp.float32),
        pltpu.VMEM((CH,), jnp.float32),
        pltpu.VMEM((DIM, CH), jnp.float32),
        pltpu.VMEM((DIM, CH), jnp.float32),
        pltpu.SemaphoreType.DMA,
        pltpu.SemaphoreType.DMA,
        pltpu.SemaphoreType.DMA,
        pltpu.SemaphoreType.DMA,
        pltpu.SemaphoreType.DMA,
        pltpu.SemaphoreType.DMA,
    ],
    compiler_params=pltpu.CompilerParams(
        use_tc_tiling_on_sc=False, needs_layout_passes=False),
)
def _sc_lookup(idx_hbm, qw_hbm, sc_hbm, out_hbm, idx_all,
               rows0, rows1, s0, s1, ot0, ot1,
               sem_r0, sem_r1, sem_s0, sem_s1, sem_o0, sem_o1):
    rows = (rows0, rows1)
    sv = (s0, s1)
    ov = (ot0, ot1)
    sem_r = (sem_r0, sem_r1)
    sem_s = (sem_s0, sem_s1)
    sem_o = (sem_o0, sem_o1)

    wid = lax.axis_index("s") * NC + lax.axis_index("c")
    base_u = wid * UNITS

    pltpu.sync_copy(idx_hbm.at[pl.ds(wid * PER_W, PER_W)], idx_all)

    riota = lax.broadcasted_iota(jnp.int32, (16,), 0)

    def start_unit(t, b):
        idx_sl = idx_all.at[pl.ds(t * CH, CH)]
        pltpu.async_copy(qw_hbm.at[idx_sl], rows[b], sem_r[b])
        pltpu.async_copy(sc_hbm.at[idx_sl], sv[b], sem_s[b])

    for b in range(NBUF):
        start_unit(b, b)

    @pl.loop(0, UNITS, step=NBUF)
    def _t(t0):
        for b in range(NBUF):
            t = t0 + b
            u = base_u + t
            h = u // BH
            bh = u % BH
            pltpu.make_async_copy(qw_hbm.at[pl.ds(0, CH)], rows[b],
                                  sem_r[b]).wait()
            pltpu.make_async_copy(sc_hbm.at[pl.ds(0, CH)], sv[b],
                                  sem_s[b]).wait()
            @pl.when(t >= NBUF)
            def _():
                pltpu.make_async_copy(qw_hbm.at[pl.ds(0, CH)], rows[b],
                                      sem_o[b]).wait()

            s16s, ridxs = [], []
            for g2 in range(CH // 16):
                s16s.append(sv[b][pl.ds(g2 * 16, 16)])
                ridxs.append(riota + (g2 * 16))

            @pl.loop(0, 16)
            def _d0(d0, b=b):
                pm = (riota + d0) & 15
                for dq in range(DIM // 16):
                    cvec = pm + (dq * 16)
                    qs = [plsc.load_gather(rows[b], [ridxs[g2], cvec])
                          for g2 in range(CH // 16)]
                    vals = [(q.astype(jnp.float32) - 8.0) * s16s[g2]
                            for g2, q in enumerate(qs)]
                    for g2 in range(CH // 16):
                        plsc.store_scatter(ov[b], [cvec, ridxs[g2]],
                                           vals[g2])

            for dh in range(DH):
                pltpu.async_copy(ov[b].at[pl.ds(dh * 8, 8)],
                                 out_hbm.at[h].at[dh].at[bh], sem_o[b])

            nt = t + NBUF

            @pl.when(nt < UNITS)
            def _():
                start_unit(nt, b)

    for b in range(NBUF):
        pltpu.make_async_copy(qw_hbm.at[pl.ds(0, CH)], rows[b],
                              sem_o[b]).wait()


def kernel(indices, qweights, scales):
    idx_flat = indices.T.reshape(N)
    out5 = _sc_lookup(idx_flat, qweights, scales)
    out = out5.transpose(2, 4, 0, 1, 3).reshape(BATCH, HIST, DIM)
    return out


# speedup vs baseline: 1.5026x; 1.5026x over previous
import functools

import jax
import jax.numpy as jnp
from jax import lax
from jax.experimental import pallas as pl
from jax.experimental.pallas import tpu as pltpu
from jax.experimental.pallas import tpu_sc as plsc

VOCAB = 1000000
DIM = 64
BATCH = 16384
HIST = 50

NC = 2
NS = 16
NW = NC * NS
N = BATCH * HIST
PER_W = N // NW
CH = 128
UNITS = PER_W // CH
BH = BATCH // CH
DH = DIM // 8
NBUF = 2

PROWS = VOCAB // 2
TCOLS = (VOCAB + 127) // 128
LASTC = TCOLS - 1
K_ITERS = (TCOLS + NW - 1) // NW


@functools.partial(
    pl.kernel,
    out_type=jax.ShapeDtypeStruct((PROWS, 2 * DIM), jnp.int32),
    mesh=plsc.VectorSubcoreMesh(
        core_axis_name="c", subcore_axis_name="s",
        num_cores=NC, num_subcores=NS),
    scratch_types=[
        pltpu.VMEM((DIM, 128), jnp.int32),
        pltpu.VMEM((DIM, 128), jnp.int32),
        pltpu.VMEM((DIM, 128), jnp.int32),
        pltpu.VMEM((DIM, 128), jnp.int32),
        pltpu.VMEM((DIM, DIM), jnp.int32),
        pltpu.SemaphoreType.DMA,
        pltpu.SemaphoreType.DMA,
        pltpu.SemaphoreType.DMA,
        pltpu.SemaphoreType.DMA,
    ],
    compiler_params=pltpu.CompilerParams(
        use_tc_tiling_on_sc=True, needs_layout_passes=False),
)
def _repack(qt_hbm, tail_hbm, p_hbm, stg0, stg1, pb0, pb1, tlb,
            sem_g0, sem_g1, sem_p0, sem_p1):
    stg = (stg0, stg1)
    pb = (pb0, pb1)
    sem_g = (sem_g0, sem_g1)
    sem_p = (sem_p0, sem_p1)

    wid = lax.axis_index("s") * NC + lax.axis_index("c")
    riota = lax.broadcasted_iota(jnp.int32, (16,), 0)
    k16 = [riota + 16 * g for g in range(4)]
    r2 = [(riota + 16 * g) * 2 for g in range(4)]

    def start_stage(k, b):
        c = wid + NW * k

        @pl.when(c < LASTC)
        def _():
            pltpu.async_copy(
                qt_hbm.at[pl.ds(0, DIM), pl.ds(c * 128, 128)],
                stg[b], sem_g[b])

        @pl.when(c == LASTC)
        def _():
            pltpu.async_copy(tail_hbm, tlb, sem_g[b])

    for b in range(NBUF):
        start_stage(b, b)

    @pl.loop(0, K_ITERS + 1, step=NBUF)
    def _k(k0):
        for b in range(NBUF):
            k = k0 + b
            c = wid + NW * k

            @pl.when(c < LASTC)
            def _():
                pltpu.make_async_copy(
                    qt_hbm.at[pl.ds(0, DIM), pl.ds(0, 128)],
                    stg[b], sem_g[b]).wait()

            @pl.when(c == LASTC)
            def _():
                pltpu.make_async_copy(tail_hbm, tlb, sem_g[b]).wait()

            @pl.when((k >= NBUF) & (c - NW * NBUF <= LASTC))
            def _():
                pltpu.make_async_copy(
                    pb[b], p_hbm.at[pl.ds(0, DIM)], sem_p[b]).wait()

            @pl.when(c < LASTC)
            def _():
                @pl.loop(0, 16)
                def _w0(w0, b=b):
                    pm = (riota + w0) & 15
                    for wq in range(8):
                        cvec = pm + (wq * 16)
                        dvec = cvec & 63
                        pp = cvec >> 6
                        qs = [plsc.load_gather(stg[b], [dvec, r2[g] + pp])
                              for g in range(4)]
                        for g in range(4):
                            plsc.store_scatter(pb[b], [k16[g], cvec], qs[g])

                pltpu.async_copy(pb[b], p_hbm.at[pl.ds(c * 64, 64)],
                                 sem_p[b])

            @pl.when(c == LASTC)
            def _():
                @pl.loop(0, 16)
                def _w0(w0, b=b):
                    pm = (riota + w0) & 15
                    for wq in range(8):
                        cvec = pm + (wq * 16)
                        dvec = cvec & 63
                        pp = cvec >> 6
                        qs = [plsc.load_gather(tlb, [r2[g] + pp, dvec])
                              for g in range(2)]
                        for g in range(2):
                            plsc.store_scatter(pb[b], [k16[g], cvec], qs[g])

                pltpu.async_copy(pb[b].at[pl.ds(0, 32)],
                                 p_hbm.at[pl.ds(c * 64, 32)], sem_p[b])

            start_stage(k + NBUF, b)

    kk = K_ITERS - 1
    b = kk % NBUF
    c = wid + NW * kk

    @pl.when(c < LASTC)
    def _():
        pltpu.make_async_copy(
            pb[b], p_hbm.at[pl.ds(0, DIM)], sem_p[b]).wait()

    @pl.when(c == LASTC)
    def _():
        pltpu.make_async_copy(
            pb[b].at[pl.ds(0, 32)], p_hbm.at[pl.ds(0, 32)],
            sem_p[b]).wait()


@functools.partial(
    pl.kernel,
    out_type=jax.ShapeDtypeStruct((HIST, DH, BH, 8, CH), jnp.float32),
    mesh=plsc.VectorSubcoreMesh(
        core_axis_name="c", subcore_axis_name="s",
        num_cores=NC, num_subcores=NS),
    scratch_types=[
        pltpu.VMEM((PER_W,), jnp.int32),
        pltpu.VMEM((PER_W,), jnp.int32),
        pltpu.VMEM((CH, 2 * DIM), jnp.int32),
        pltpu.VMEM((CH, 2 * DIM), jnp.int32),
        pltpu.VMEM((CH,), jnp.float32),
        pltpu.VMEM((CH,), jnp.float32),
        pltpu.VMEM((DIM, CH), jnp.float32),
        pltpu.VMEM((DIM, CH), jnp.float32),
        pltpu.SemaphoreType.DMA,
        pltpu.SemaphoreType.DMA,
        pltpu.SemaphoreType.DMA,
        pltpu.SemaphoreType.DMA,
        pltpu.SemaphoreType.DMA,
        pltpu.SemaphoreType.DMA,
    ],
    compiler_params=pltpu.CompilerParams(
        use_tc_tiling_on_sc=True, needs_layout_passes=False),
)
def _lookup(idx_hbm, pidx_hbm, qw2_hbm, sc_hbm, out_hbm, idx_all, pidx_all,
            rows0, rows1, s0, s1, ot0, ot1,
            sem_r0, sem_r1, sem_s0, sem_s1, sem_o0, sem_o1):
    rows = (rows0, rows1)
    sv = (s0, s1)
    ov = (ot0, ot1)
    sem_r = (sem_r0, sem_r1)
    sem_s = (sem_s0, sem_s1)
    sem_o = (sem_o0, sem_o1)

    wid = lax.axis_index("s") * NC + lax.axis_index("c")
    base_u = wid * UNITS

    pltpu.sync_copy(idx_hbm.at[pl.ds(wid * PER_W, PER_W)], idx_all)
    pltpu.sync_copy(pidx_hbm.at[pl.ds(wid * PER_W, PER_W)], pidx_all)

    riota = lax.broadcasted_iota(jnp.int32, (16,), 0)

    def start_unit(t, b):
        pltpu.async_copy(qw2_hbm.at[pidx_all.at[pl.ds(t * CH, CH)]],
                         rows[b], sem_r[b])
        pltpu.async_copy(sc_hbm.at[idx_all.at[pl.ds(t * CH, CH)]],
                         sv[b], sem_s[b])

    for b in range(NBUF):
        start_unit(b, b)

    @pl.loop(0, UNITS, step=NBUF)
    def _t(t0):
        for b in range(NBUF):
            t = t0 + b
            u = base_u + t
            h = u // BH
            bh = u % BH
            pltpu.make_async_copy(qw2_hbm.at[pl.ds(0, CH)], rows[b],
                                  sem_r[b]).wait()
            pltpu.make_async_copy(sc_hbm.at[pl.ds(0, CH)], sv[b],
                                  sem_s[b]).wait()
            @pl.when(t >= NBUF)
            def _():
                for dh in range(DH):
                    pltpu.make_async_copy(
                        ov[b].at[pl.ds(dh * 8, 8)],
                        out_hbm.at[0].at[0].at[0], sem_o[b]).wait()

            s16s, ridxs, w64s = [], [], []
            for g2 in range(CH // 16):
                s16s.append(sv[b][pl.ds(g2 * 16, 16)])
                ridxs.append(riota + (g2 * 16))
                iv = idx_all[pl.ds(t * CH + g2 * 16, 16)]
                w64s.append((iv & 1) << 6)

            @pl.loop(0, 16)
            def _d0(d0, b=b):
                pm = (riota + d0) & 15
                for dq in range(DIM // 16):
                    cvec = pm + (dq * 16)
                    qs = [plsc.load_gather(rows[b],
                                           [ridxs[g2], cvec + w64s[g2]])
                          for g2 in range(CH // 16)]
                    vals = [(q.astype(jnp.float32) - 8.0) * s16s[g2]
                            for g2, q in enumerate(qs)]
                    for g2 in range(CH // 16):
                        plsc.store_scatter(ov[b], [cvec, ridxs[g2]],
                                           vals[g2])

            for dh in range(DH):
                pltpu.async_copy(ov[b].at[pl.ds(dh * 8, 8)],
                                 out_hbm.at[h].at[dh].at[bh], sem_o[b])

            nt = t + NBUF

            @pl.when(nt < UNITS)
            def _():
                start_unit(nt, b)

    for b in range(NBUF):
        for dh in range(DH):
            pltpu.make_async_copy(ov[b].at[pl.ds(dh * 8, 8)],
                                  out_hbm.at[0].at[0].at[0], sem_o[b]).wait()


def kernel(indices, qweights, scales):
    tail = qweights[VOCAB - DIM:, :]
    packed = _repack(qweights.T, tail)
    idx_flat = indices.T.reshape(N)
    out5 = _lookup(idx_flat, idx_flat >> 1, packed, scales)
    out = out5.transpose(2, 4, 0, 1, 3).reshape(BATCH, HIST, DIM)
    return out
